# Initial kernel scaffold; baseline (speedup 1.0000x reference)
import jax
import jax.numpy as jnp
from jax.experimental import pallas as pl


def kernel(text_pooled, x, edge_index, batch, Wt, bt, temp, ln1_g, ln1_b, ln2_g, ln2_b, W1, b1, W2, b2, W3, b3, Wm1, bm1, Wm2, bm2, Wm3, bm3):
    raise NotImplementedError("write your pallas kernel here")


# speedup vs baseline: 13.6072x; 13.6072x over previous
import functools

import jax
import jax.numpy as jnp
from jax import lax
from jax.experimental import pallas as pl
from jax.experimental.pallas import tpu as pltpu
from jax.experimental.pallas import tpu_sc as plsc

N = 10000
E = 320000
DF = 128
HF = DF // 2
GH = 128
NINP = 768
NHID = 256
NOUT = 256
B = 128

NC = 2
NS = 16
NW = NC * NS

CH = 80
NCHD = (E // NW) // CH
NCHS = (E // NS) // CH
RPT = 624
RTAIL = N - NS * RPT

_mesh = plsc.VectorSubcoreMesh(
    core_axis_name="c", subcore_axis_name="s", num_cores=NC, num_subcores=NS)


@functools.partial(
    pl.kernel,
    out_type=jax.ShapeDtypeStruct((NC, N, 8), jnp.float32),
    mesh=_mesh,
    scratch_types=[
        pltpu.VMEM((NCHD, CH), jnp.int32),
        pltpu.VMEM((CH, 8), jnp.float32),
        pltpu.VMEM_SHARED((N, 8), jnp.float32),
    ],
)
def _sc_deg(dst_hbm, zeros8_hbm, ones8_hbm, out_hbm, dst_v, ones_v, acc):
  c = lax.axis_index("c")
  s = lax.axis_index("s")
  w = c * NS + s
  pltpu.sync_copy(dst_hbm.at[w], dst_v)
  pltpu.sync_copy(ones8_hbm, ones_v)

  @pl.when(s == 0)
  def _():
    pltpu.sync_copy(zeros8_hbm, acc)

  plsc.subcore_barrier()

  def body(j, carry):
    pltpu.sync_copy(ones_v, acc.at[dst_v.at[j]], add=True)
    return carry

  lax.fori_loop(0, NCHD, body, 0)
  plsc.subcore_barrier()
  pltpu.sync_copy(acc.at[pl.ds(s * RPT, RPT)],
                  out_hbm.at[c, pl.ds(s * RPT, RPT)])

  @pl.when(s == 0)
  def _():
    pltpu.sync_copy(acc.at[pl.ds(NS * RPT, RTAIL)],
                    out_hbm.at[c, pl.ds(NS * RPT, RTAIL)])


@functools.partial(
    pl.kernel,
    out_type=jax.ShapeDtypeStruct((NC, N, HF), jnp.float32),
    mesh=_mesh,
    scratch_types=[
        pltpu.VMEM((NCHS, CH), jnp.int32),
        pltpu.VMEM((NCHS, CH), jnp.int32),
        pltpu.VMEM((CH, HF), jnp.float32),
        pltpu.VMEM((CH, HF), jnp.float32),
        pltpu.VMEM_SHARED((N, HF), jnp.float32),
        pltpu.SemaphoreType.DMA,
        pltpu.SemaphoreType.DMA,
    ],
    compiler_params=pltpu.CompilerParams(use_tc_tiling_on_sc=False),
)
def _sc_scatter(h_hbm, src_hbm, dst_hbm, zeros_hbm, out_hbm,
                src_v, dst_v, buf0, buf1, acc, sem0, sem1):
  c = lax.axis_index("c")
  s = lax.axis_index("s")
  pltpu.sync_copy(src_hbm.at[c, s], src_v)
  pltpu.sync_copy(dst_hbm.at[s], dst_v)

  @pl.when(s == 0)
  def _():
    pltpu.sync_copy(zeros_hbm, acc)

  plsc.subcore_barrier()

  def start(j, buf, sem):
    pltpu.make_async_copy(h_hbm.at[src_v.at[j]], buf, sem).start()

  def wait(j, buf, sem):
    pltpu.make_async_copy(h_hbm.at[src_v.at[j]], buf, sem).wait()

  start(0, buf0, sem0)

  def pair(p, carry):
    j0 = 2 * p
    j1 = j0 + 1
    wait(j0, buf0, sem0)
    start(j1, buf1, sem1)
    pltpu.sync_copy(buf0, acc.at[dst_v.at[j0]], add=True)
    wait(j1, buf1, sem1)

    @pl.when(j1 < NCHS - 1)
    def _():
      start(j0 + 2, buf0, sem0)

    pltpu.sync_copy(buf1, acc.at[dst_v.at[j1]], add=True)
    return carry

  lax.fori_loop(0, NCHS // 2, pair, 0)

  plsc.subcore_barrier()
  pltpu.sync_copy(acc.at[pl.ds(s * RPT, RPT)],
                  out_hbm.at[c, pl.ds(s * RPT, RPT)])

  @pl.when(s == 0)
  def _():
    pltpu.sync_copy(acc.at[pl.ds(NS * RPT, RTAIL)],
                    out_hbm.at[c, pl.ds(NS * RPT, RTAIL)])


_MMB = 1000


def _mm1_body(x_ref, w_ref, degp_ref, lo_ref, hi_ref, dinv_ref):
  deg = jnp.sum(degp_ref[...], axis=1, keepdims=True) + 1.0
  dinv = lax.rsqrt(deg)
  h = jnp.dot(x_ref[...], w_ref[...], preferred_element_type=jnp.float32)
  hp = h * dinv
  lo_ref[...] = hp[:, :HF]
  hi_ref[...] = hp[:, HF:]
  dinv_ref[...] = dinv


def _mm1(x, w, degp):
  return pl.pallas_call(
      _mm1_body,
      grid=(N // _MMB,),
      in_specs=[
          pl.BlockSpec((_MMB, DF), lambda i: (i, 0)),
          pl.BlockSpec((DF, DF), lambda i: (0, 0)),
          pl.BlockSpec((_MMB, 16), lambda i: (i, 0)),
      ],
      out_specs=[
          pl.BlockSpec((_MMB, HF), lambda i: (i, 0)),
          pl.BlockSpec((_MMB, HF), lambda i: (i, 0)),
          pl.BlockSpec((_MMB, 1), lambda i: (i, 0)),
      ],
      out_shape=[
          jax.ShapeDtypeStruct((N, HF), jnp.float32),
          jax.ShapeDtypeStruct((N, HF), jnp.float32),
          jax.ShapeDtypeStruct((N, 1), jnp.float32),
      ],
  )(x, w, degp)


def _mm_mid_body(alo_ref, ahi_ref, hlo_ref, hhi_ref, dinv_ref, b_ref, w_ref,
                 lo_ref, hi_ref):
  dinv = dinv_ref[...]
  s = jnp.concatenate([alo_ref[...] + hlo_ref[...],
                       ahi_ref[...] + hhi_ref[...]], axis=1)
  xl = jnp.maximum(dinv * s + b_ref[...], 0.0)
  h = jnp.dot(xl, w_ref[...], preferred_element_type=jnp.float32)
  hp = h * dinv
  lo_ref[...] = hp[:, :HF]
  hi_ref[...] = hp[:, HF:]


def _mm_mid(alo, ahi, hlo, hhi, dinv, b, w):
  return pl.pallas_call(
      _mm_mid_body,
      grid=(N // _MMB,),
      in_specs=[
          pl.BlockSpec((_MMB, HF), lambda i: (i, 0)),
          pl.BlockSpec((_MMB, HF), lambda i: (i, 0)),
          pl.BlockSpec((_MMB, HF), lambda i: (i, 0)),
          pl.BlockSpec((_MMB, HF), lambda i: (i, 0)),
          pl.BlockSpec((_MMB, 1), lambda i: (i, 0)),
          pl.BlockSpec((1, DF), lambda i: (0, 0)),
          pl.BlockSpec((DF, DF), lambda i: (0, 0)),
      ],
      out_specs=[
          pl.BlockSpec((_MMB, HF), lambda i: (i, 0)),
          pl.BlockSpec((_MMB, HF), lambda i: (i, 0)),
      ],
      out_shape=[
          jax.ShapeDtypeStruct((N, HF), jnp.float32),
          jax.ShapeDtypeStruct((N, HF), jnp.float32),
      ],
  )(alo, ahi, hlo, hhi, dinv, b, w)


_PB = 400
_NPB = N // _PB


def _ln(h, g, b):
  mu = jnp.mean(h, axis=-1, keepdims=True)
  var = jnp.mean((h - mu) * (h - mu), axis=-1, keepdims=True)
  return (h - mu) / jnp.sqrt(var + 1e-5) * g + b


def _final_body(alo_ref, ahi_ref, hlo_ref, hhi_ref, dinv_ref, b3_ref,
                batch_ref,
                wm1_ref, bm1_ref, wm2_ref, bm2_ref, wm3_ref, bm3_ref,
                ln1g_ref, ln1b_ref, ln2g_ref, ln2b_ref,
                text_ref, wt_ref, bt_ref, temp_ref,
                otx_ref, og_ref, psum):
  i = pl.program_id(0)

  @pl.when(i == 0)
  def _():
    psum[...] = jnp.zeros_like(psum)

  @pl.when(i < _NPB)
  def _():
    s = jnp.concatenate([alo_ref[...] + hlo_ref[...],
                         ahi_ref[...] + hhi_ref[...]], axis=1)
    out3 = dinv_ref[...] * s + b3_ref[...]
    bidx = batch_ref[0, 0, :]
    onehot = (bidx[:, None] == lax.broadcasted_iota(
        jnp.int32, (_PB, B), 1)).astype(jnp.float32)
    cat = jnp.concatenate(
        [out3, jnp.ones((_PB, DF), jnp.float32)], axis=1)
    psum[...] += lax.dot_general(
        onehot, cat, (((0,), (0,)), ((), ())),
        preferred_element_type=jnp.float32)

  @pl.when(i == _NPB)
  def _():
    scale = jnp.exp(temp_ref[0, 0])
    sums = psum[:, :DF]
    cnts = psum[:, DF:DF + 1]
    g = sums / jnp.maximum(cnts, 1.0)
    g = jnp.maximum(jnp.dot(g, wm1_ref[...],
                            preferred_element_type=jnp.float32)
                    + bm1_ref[...], 0.0)
    g = jnp.maximum(jnp.dot(g, wm2_ref[...],
                            preferred_element_type=jnp.float32)
                    + bm2_ref[...], 0.0)
    g = jnp.dot(g, wm3_ref[...], preferred_element_type=jnp.float32) \
        + bm3_ref[...]
    og_ref[...] = _ln(g, ln1g_ref[...], ln1b_ref[...]) * scale
    tx = jnp.dot(text_ref[...], wt_ref[...],
                 preferred_element_type=jnp.float32) + bt_ref[...]
    otx_ref[...] = _ln(tx, ln2g_ref[...], ln2b_ref[...]) * scale


def _final(alo, ahi, hlo, hhi, dinv, b3, batch3, wm1, bm1, wm2, bm2, wm3,
           bm3, ln1g, ln1b, ln2g, ln2b, text, wt, bt, temp):
  row = lambda i: (jnp.minimum(i, _NPB - 1), 0)
  full = lambda i: (0, 0)
  return pl.pallas_call(
      _final_body,
      grid=(_NPB + 1,),
      in_specs=[
          pl.BlockSpec((_PB, HF), row),
          pl.BlockSpec((_PB, HF), row),
          pl.BlockSpec((_PB, HF), row),
          pl.BlockSpec((_PB, HF), row),
          pl.BlockSpec((_PB, 1), row),
          pl.BlockSpec((1, DF), full),
          pl.BlockSpec((1, 1, _PB), lambda i: (jnp.minimum(i, _NPB - 1), 0, 0)),
          pl.BlockSpec((DF, NHID), full),
          pl.BlockSpec((1, NHID), full),
          pl.BlockSpec((NHID, NHID), full),
          pl.BlockSpec((1, NHID), full),
          pl.BlockSpec((NHID, NOUT), full),
          pl.BlockSpec((1, NOUT), full),
          pl.BlockSpec((1, NOUT), full),
          pl.BlockSpec((1, NOUT), full),
          pl.BlockSpec((1, NOUT), full),
          pl.BlockSpec((1, NOUT), full),
          pl.BlockSpec((B, NINP), full),
          pl.BlockSpec((NINP, NOUT), full),
          pl.BlockSpec((1, NOUT), full),
          pl.BlockSpec((1, 1), full),
      ],
      out_specs=[
          pl.BlockSpec((B, NOUT), full),
          pl.BlockSpec((B, NOUT), full),
      ],
      out_shape=[
          jax.ShapeDtypeStruct((B, NOUT), jnp.float32),
          jax.ShapeDtypeStruct((B, NOUT), jnp.float32),
      ],
      scratch_shapes=[pltpu.VMEM((B, DF + DF), jnp.float32)],
  )(alo, ahi, hlo, hhi, dinv, b3, batch3, wm1, bm1, wm2, bm2, wm3, bm3,
    ln1g, ln1b, ln2g, ln2b, text, wt, bt, temp)


def kernel(text_pooled, x, edge_index, batch, Wt, bt, temp,
           ln1_g, ln1_b, ln2_g, ln2_b, W1, b1, W2, b2, W3, b3,
           Wm1, bm1, Wm2, bm2, Wm3, bm3):
  src = edge_index[0]
  dst = edge_index[1]
  dst_deg = dst.reshape(NW, NCHD, CH)
  src_sc = jnp.stack([src, src + N]).reshape(NC, NS, NCHS, CH)
  dst_sc = dst.reshape(NS, NCHS, CH)

  zeros8 = jnp.zeros((N, 8), jnp.float32)
  ones8 = jnp.ones((CH, 8), jnp.float32)
  zeros = jnp.zeros((N, HF), jnp.float32)

  degp = _sc_deg(dst_deg, zeros8, ones8)
  degp2 = degp.transpose(1, 0, 2).reshape(N, 16)

  def table(lo, hi):
    return jnp.concatenate([lo, hi], axis=0)

  h1lo, h1hi, dinv = _mm1(x, W1, degp2)
  a1 = _sc_scatter(table(h1lo, h1hi), src_sc, dst_sc, zeros)
  h2lo, h2hi = _mm_mid(a1[0], a1[1], h1lo, h1hi, dinv,
                       b1.reshape(1, DF), W2)
  a2 = _sc_scatter(table(h2lo, h2hi), src_sc, dst_sc, zeros)
  h3lo, h3hi = _mm_mid(a2[0], a2[1], h2lo, h2hi, dinv,
                       b2.reshape(1, DF), W3)
  a3 = _sc_scatter(table(h3lo, h3hi), src_sc, dst_sc, zeros)

  batch3 = batch.reshape(_NPB, 1, _PB)
  tx, g = _final(a3[0], a3[1], h3lo, h3hi, dinv, b3.reshape(1, DF), batch3,
                 Wm1, bm1.reshape(1, NHID), Wm2, bm2.reshape(1, NHID),
                 Wm3, bm3.reshape(1, NOUT),
                 ln1_g.reshape(1, NOUT), ln1_b.reshape(1, NOUT),
                 ln2_g.reshape(1, NOUT), ln2_b.reshape(1, NOUT),
                 text_pooled, Wt, bt.reshape(1, NOUT), temp.reshape(1, 1))
  return tx, g


# speedup vs baseline: 21.4055x; 1.5731x over previous
import functools

import jax
import jax.numpy as jnp
from jax import lax
from jax.experimental import pallas as pl
from jax.experimental.pallas import tpu as pltpu
from jax.experimental.pallas import tpu_sc as plsc

N = 10000
E = 320000
DF = 128
HF = DF // 2
GH = 128
NINP = 768
NHID = 256
NOUT = 256
B = 128

NC = 2
NS = 16
NW = NC * NS

CH = 125
NCHD = (E // NW) // CH
NCHS = (E // NS) // CH
PD = 4
RPT = 624
RTAIL = N - NS * RPT

_mesh = plsc.VectorSubcoreMesh(
    core_axis_name="c", subcore_axis_name="s", num_cores=NC, num_subcores=NS)


@functools.partial(
    pl.kernel,
    out_type=jax.ShapeDtypeStruct((NC, N, 8), jnp.float32),
    mesh=_mesh,
    scratch_types=[
        pltpu.VMEM((NCHD, CH), jnp.int32),
        pltpu.VMEM((CH, 8), jnp.float32),
        pltpu.VMEM_SHARED((N, 8), jnp.float32),
    ],
)
def _sc_deg(dst_hbm, zeros8_hbm, ones8_hbm, out_hbm, dst_v, ones_v, acc):
  c = lax.axis_index("c")
  s = lax.axis_index("s")
  w = c * NS + s
  pltpu.sync_copy(dst_hbm.at[w], dst_v)
  pltpu.sync_copy(ones8_hbm, ones_v)

  @pl.when(s == 0)
  def _():
    pltpu.sync_copy(zeros8_hbm, acc)

  plsc.subcore_barrier()

  def body(j, carry):
    pltpu.sync_copy(ones_v, acc.at[dst_v.at[j]], add=True)
    return carry

  lax.fori_loop(0, NCHD, body, 0)
  plsc.subcore_barrier()
  pltpu.sync_copy(acc.at[pl.ds(s * RPT, RPT)],
                  out_hbm.at[c, pl.ds(s * RPT, RPT)])

  @pl.when(s == 0)
  def _():
    pltpu.sync_copy(acc.at[pl.ds(NS * RPT, RTAIL)],
                    out_hbm.at[c, pl.ds(NS * RPT, RTAIL)])


@functools.partial(
    pl.kernel,
    out_type=jax.ShapeDtypeStruct((NC, N, HF), jnp.float32),
    mesh=_mesh,
    scratch_types=[
        pltpu.VMEM((NCHS, CH), jnp.int32),
        pltpu.VMEM((NCHS, CH), jnp.int32),
        [pltpu.VMEM((CH, HF), jnp.float32) for _ in range(PD)],
        [pltpu.SemaphoreType.DMA for _ in range(PD)],
        [pltpu.SemaphoreType.DMA for _ in range(PD)],
        pltpu.VMEM_SHARED((N, HF), jnp.float32),
    ],
    compiler_params=pltpu.CompilerParams(use_tc_tiling_on_sc=False),
)
def _sc_scatter(h_hbm, src_hbm, dst_hbm, zeros_hbm, out_hbm,
                src_v, dst_v, bufs, gsems, ssems, acc):
  c = lax.axis_index("c")
  s = lax.axis_index("s")
  pltpu.sync_copy(src_hbm.at[c, s], src_v)
  pltpu.sync_copy(dst_hbm.at[s], dst_v)

  @pl.when(s == 0)
  def _():
    pltpu.sync_copy(zeros_hbm, acc)

  plsc.subcore_barrier()

  def start_gather(j, k):
    pltpu.make_async_copy(h_hbm.at[src_v.at[j]], bufs[k], gsems[k]).start()

  def wait_gather(j, k):
    pltpu.make_async_copy(h_hbm.at[src_v.at[j]], bufs[k], gsems[k]).wait()

  def start_scatter(j, k):
    pltpu.async_copy(bufs[k], acc.at[dst_v.at[j]], ssems[k], add=True)

  def wait_scatter(j, k):
    pltpu.make_async_copy(bufs[k], acc.at[dst_v.at[j]], ssems[k]).wait()

  for k in range(PD):
    start_gather(k, k)

  NG = NCHS // PD

  def group(g, carry):
    j = PD * g
    for k in range(PD):
      wait_gather(j + k, k)
      start_scatter(j + k, k)
    for k in range(PD):
      wait_scatter(j + k, k)

      @pl.when(g < NG - 1)
      def _():
        start_gather(j + PD + k, k)

    return carry

  lax.fori_loop(0, NG, group, 0)

  plsc.subcore_barrier()
  pltpu.sync_copy(acc.at[pl.ds(s * RPT, RPT)],
                  out_hbm.at[c, pl.ds(s * RPT, RPT)])

  @pl.when(s == 0)
  def _():
    pltpu.sync_copy(acc.at[pl.ds(NS * RPT, RTAIL)],
                    out_hbm.at[c, pl.ds(NS * RPT, RTAIL)])


_MMB = 1000


def _mm1_body(x_ref, w_ref, degp_ref, lo_ref, hi_ref, dinv_ref):
  deg = jnp.sum(degp_ref[...], axis=1, keepdims=True) + 1.0
  dinv = lax.rsqrt(deg)
  h = jnp.dot(x_ref[...], w_ref[...], preferred_element_type=jnp.float32)
  hp = h * dinv
  lo_ref[...] = hp[:, :HF]
  hi_ref[...] = hp[:, HF:]
  dinv_ref[...] = dinv


def _mm1(x, w, degp):
  return pl.pallas_call(
      _mm1_body,
      grid=(N // _MMB,),
      in_specs=[
          pl.BlockSpec((_MMB, DF), lambda i: (i, 0)),
          pl.BlockSpec((DF, DF), lambda i: (0, 0)),
          pl.BlockSpec((_MMB, 16), lambda i: (i, 0)),
      ],
      out_specs=[
          pl.BlockSpec((_MMB, HF), lambda i: (i, 0)),
          pl.BlockSpec((_MMB, HF), lambda i: (i, 0)),
          pl.BlockSpec((_MMB, 1), lambda i: (i, 0)),
      ],
      out_shape=[
          jax.ShapeDtypeStruct((N, HF), jnp.float32),
          jax.ShapeDtypeStruct((N, HF), jnp.float32),
          jax.ShapeDtypeStruct((N, 1), jnp.float32),
      ],
  )(x, w, degp)


def _mm_mid_body(alo_ref, ahi_ref, hlo_ref, hhi_ref, dinv_ref, b_ref, w_ref,
                 lo_ref, hi_ref):
  dinv = dinv_ref[...]
  s = jnp.concatenate([alo_ref[...] + hlo_ref[...],
                       ahi_ref[...] + hhi_ref[...]], axis=1)
  xl = jnp.maximum(dinv * s + b_ref[...], 0.0)
  h = jnp.dot(xl, w_ref[...], preferred_element_type=jnp.float32)
  hp = h * dinv
  lo_ref[...] = hp[:, :HF]
  hi_ref[...] = hp[:, HF:]


def _mm_mid(alo, ahi, hlo, hhi, dinv, b, w):
  return pl.pallas_call(
      _mm_mid_body,
      grid=(N // _MMB,),
      in_specs=[
          pl.BlockSpec((_MMB, HF), lambda i: (i, 0)),
          pl.BlockSpec((_MMB, HF), lambda i: (i, 0)),
          pl.BlockSpec((_MMB, HF), lambda i: (i, 0)),
          pl.BlockSpec((_MMB, HF), lambda i: (i, 0)),
          pl.BlockSpec((_MMB, 1), lambda i: (i, 0)),
          pl.BlockSpec((1, DF), lambda i: (0, 0)),
          pl.BlockSpec((DF, DF), lambda i: (0, 0)),
      ],
      out_specs=[
          pl.BlockSpec((_MMB, HF), lambda i: (i, 0)),
          pl.BlockSpec((_MMB, HF), lambda i: (i, 0)),
      ],
      out_shape=[
          jax.ShapeDtypeStruct((N, HF), jnp.float32),
          jax.ShapeDtypeStruct((N, HF), jnp.float32),
      ],
  )(alo, ahi, hlo, hhi, dinv, b, w)


_PB = 400
_NPB = N // _PB


def _ln(h, g, b):
  mu = jnp.mean(h, axis=-1, keepdims=True)
  var = jnp.mean((h - mu) * (h - mu), axis=-1, keepdims=True)
  return (h - mu) / jnp.sqrt(var + 1e-5) * g + b


def _final_body(alo_ref, ahi_ref, hlo_ref, hhi_ref, dinv_ref, b3_ref,
                batch_ref,
                wm1_ref, bm1_ref, wm2_ref, bm2_ref, wm3_ref, bm3_ref,
                ln1g_ref, ln1b_ref, ln2g_ref, ln2b_ref,
                text_ref, wt_ref, bt_ref, temp_ref,
                otx_ref, og_ref, psum):
  i = pl.program_id(0)

  @pl.when(i == 0)
  def _():
    psum[...] = jnp.zeros_like(psum)

  @pl.when(i < _NPB)
  def _():
    s = jnp.concatenate([alo_ref[...] + hlo_ref[...],
                         ahi_ref[...] + hhi_ref[...]], axis=1)
    out3 = dinv_ref[...] * s + b3_ref[...]
    bidx = batch_ref[0, 0, :]
    onehot = (bidx[:, None] == lax.broadcasted_iota(
        jnp.int32, (_PB, B), 1)).astype(jnp.float32)
    cat = jnp.concatenate(
        [out3, jnp.ones((_PB, DF), jnp.float32)], axis=1)
    psum[...] += lax.dot_general(
        onehot, cat, (((0,), (0,)), ((), ())),
        preferred_element_type=jnp.float32)

  @pl.when(i == _NPB)
  def _():
    scale = jnp.exp(temp_ref[0, 0])
    sums = psum[:, :DF]
    cnts = psum[:, DF:DF + 1]
    g = sums / jnp.maximum(cnts, 1.0)
    g = jnp.maximum(jnp.dot(g, wm1_ref[...],
                            preferred_element_type=jnp.float32)
                    + bm1_ref[...], 0.0)
    g = jnp.maximum(jnp.dot(g, wm2_ref[...],
                            preferred_element_type=jnp.float32)
                    + bm2_ref[...], 0.0)
    g = jnp.dot(g, wm3_ref[...], preferred_element_type=jnp.float32) \
        + bm3_ref[...]
    og_ref[...] = _ln(g, ln1g_ref[...], ln1b_ref[...]) * scale
    tx = jnp.dot(text_ref[...], wt_ref[...],
                 preferred_element_type=jnp.float32) + bt_ref[...]
    otx_ref[...] = _ln(tx, ln2g_ref[...], ln2b_ref[...]) * scale


def _final(alo, ahi, hlo, hhi, dinv, b3, batch3, wm1, bm1, wm2, bm2, wm3,
           bm3, ln1g, ln1b, ln2g, ln2b, text, wt, bt, temp):
  row = lambda i: (jnp.minimum(i, _NPB - 1), 0)
  full = lambda i: (0, 0)
  return pl.pallas_call(
      _final_body,
      grid=(_NPB + 1,),
      in_specs=[
          pl.BlockSpec((_PB, HF), row),
          pl.BlockSpec((_PB, HF), row),
          pl.BlockSpec((_PB, HF), row),
          pl.BlockSpec((_PB, HF), row),
          pl.BlockSpec((_PB, 1), row),
          pl.BlockSpec((1, DF), full),
          pl.BlockSpec((1, 1, _PB), lambda i: (jnp.minimum(i, _NPB - 1), 0, 0)),
          pl.BlockSpec((DF, NHID), full),
          pl.BlockSpec((1, NHID), full),
          pl.BlockSpec((NHID, NHID), full),
          pl.BlockSpec((1, NHID), full),
          pl.BlockSpec((NHID, NOUT), full),
          pl.BlockSpec((1, NOUT), full),
          pl.BlockSpec((1, NOUT), full),
          pl.BlockSpec((1, NOUT), full),
          pl.BlockSpec((1, NOUT), full),
          pl.BlockSpec((1, NOUT), full),
          pl.BlockSpec((B, NINP), full),
          pl.BlockSpec((NINP, NOUT), full),
          pl.BlockSpec((1, NOUT), full),
          pl.BlockSpec((1, 1), full),
      ],
      out_specs=[
          pl.BlockSpec((B, NOUT), full),
          pl.BlockSpec((B, NOUT), full),
      ],
      out_shape=[
          jax.ShapeDtypeStruct((B, NOUT), jnp.float32),
          jax.ShapeDtypeStruct((B, NOUT), jnp.float32),
      ],
      scratch_shapes=[pltpu.VMEM((B, DF + DF), jnp.float32)],
  )(alo, ahi, hlo, hhi, dinv, b3, batch3, wm1, bm1, wm2, bm2, wm3, bm3,
    ln1g, ln1b, ln2g, ln2b, text, wt, bt, temp)


def kernel(text_pooled, x, edge_index, batch, Wt, bt, temp,
           ln1_g, ln1_b, ln2_g, ln2_b, W1, b1, W2, b2, W3, b3,
           Wm1, bm1, Wm2, bm2, Wm3, bm3):
  src = edge_index[0]
  dst = edge_index[1]
  dst_deg = dst.reshape(NW, NCHD, CH)
  src_sc = jnp.stack([src, src + N]).reshape(NC, NS, NCHS, CH)
  dst_sc = dst.reshape(NS, NCHS, CH)

  zeros8 = jnp.zeros((N, 8), jnp.float32)
  ones8 = jnp.ones((CH, 8), jnp.float32)
  zeros = jnp.zeros((N, HF), jnp.float32)

  degp = _sc_deg(dst_deg, zeros8, ones8)
  degp2 = degp.transpose(1, 0, 2).reshape(N, 16)

  def table(lo, hi):
    return jnp.concatenate([lo, hi], axis=0)

  h1lo, h1hi, dinv = _mm1(x, W1, degp2)
  a1 = _sc_scatter(table(h1lo, h1hi), src_sc, dst_sc, zeros)
  h2lo, h2hi = _mm_mid(a1[0], a1[1], h1lo, h1hi, dinv,
                       b1.reshape(1, DF), W2)
  a2 = _sc_scatter(table(h2lo, h2hi), src_sc, dst_sc, zeros)
  h3lo, h3hi = _mm_mid(a2[0], a2[1], h2lo, h2hi, dinv,
                       b2.reshape(1, DF), W3)
  a3 = _sc_scatter(table(h3lo, h3hi), src_sc, dst_sc, zeros)

  batch3 = batch.reshape(_NPB, 1, _PB)
  tx, g = _final(a3[0], a3[1], h3lo, h3hi, dinv, b3.reshape(1, DF), batch3,
                 Wm1, bm1.reshape(1, NHID), Wm2, bm2.reshape(1, NHID),
                 Wm3, bm3.reshape(1, NOUT),
                 ln1_g.reshape(1, NOUT), ln1_b.reshape(1, NOUT),
                 ln2_g.reshape(1, NOUT), ln2_b.reshape(1, NOUT),
                 text_pooled, Wt, bt.reshape(1, NOUT), temp.reshape(1, 1))
  return tx, g
